# baseline (speedup 1.0000x reference)
import functools

import jax
import jax.numpy as jnp
from jax import lax
from jax.experimental import pallas as pl
from jax.experimental.pallas import tpu as pltpu
from jax.experimental.pallas import tpu_sc as plsc

B = 128
V = 32768
L = 16
NC = 2
NS = 16
NW = NC * NS
ROWS_PER_W = B // NW
U = 8
CHUNK = U * L
VBLK = 4096


def _tc_max_body(x_ref, o_ref, acc_ref):
    i = pl.program_id(0)
    m = jnp.max(x_ref[...].reshape(B, VBLK // 128, 128), axis=1)

    @pl.when(i == 0)
    def _():
        acc_ref[...] = m

    @pl.when(i > 0)
    def _():
        acc_ref[...] = jnp.maximum(acc_ref[...], m)

    @pl.when(i == V // VBLK - 1)
    def _():
        o_ref[...] = jnp.max(acc_ref[...], axis=1)[None, :]


_tc_row_max = pl.pallas_call(
    _tc_max_body,
    grid=(V // VBLK,),
    in_specs=[pl.BlockSpec((B, VBLK), lambda i: (0, i))],
    out_specs=pl.BlockSpec((1, B), lambda i: (0, 0)),
    out_shape=jax.ShapeDtypeStruct((1, B), jnp.float32),
    scratch_shapes=[pltpu.VMEM((B, 128), jnp.float32)],
)


def _butterfly(v, op):
    lane = lax.iota(jnp.int32, L)
    for k in (8, 4, 2, 1):
        v = op(v, v.at[lane ^ k].get(mode="promise_in_bounds"))
    return v


def _sumexp_accs(buf, base, n, row_max, init):
    @plsc.parallel_loop(0, n, CHUNK, unroll=4, carry=init)
    def ss(off, ss):
        return tuple(
            ss[u] + jnp.exp(buf[pl.ds(base + off + u * L, L)] - row_max)
            for u in range(U)
        )

    return ss


def _accs_total(ss):
    s = ss[0]
    for u in range(1, U):
        s = s + ss[u]
    return _butterfly(s, jnp.add)


def _zero_accs():
    return tuple(jnp.zeros((L,), jnp.float32) for _ in range(U))


@functools.partial(
    pl.kernel,
    mesh=plsc.VectorSubcoreMesh(core_axis_name="c", subcore_axis_name="s"),
    out_type=jax.ShapeDtypeStruct((NW, L), jnp.float32),
    scratch_types=[
        pltpu.VMEM((2 * V,), jnp.float32),
        pltpu.VMEM((B,), jnp.float32),
        pltpu.VMEM((L,), jnp.float32),
        pltpu.SemaphoreType.DMA((4,)),
    ],
)
def _sc_top1(x_hbm, mx_hbm, out_hbm, buf, mx_v, out_buf, sems):
    cid = lax.axis_index("c")
    sid = lax.axis_index("s")
    wid = sid * NC + cid
    base_row = wid * ROWS_PER_W
    H = V // 2

    pltpu.make_async_copy(
        x_hbm.at[base_row, pl.ds(0, H)], buf.at[pl.ds(0, H)], sems.at[2]
    ).start()
    pltpu.make_async_copy(
        x_hbm.at[base_row, pl.ds(H, H)], buf.at[pl.ds(H, H)], sems.at[3]
    ).start()
    pltpu.make_async_copy(
        x_hbm.at[base_row + 1], buf.at[pl.ds(V, V)], sems.at[1]
    ).start()
    pltpu.sync_copy(mx_hbm.at[0], mx_v)

    lane = lax.iota(jnp.int32, L)
    mx_chunk = mx_v[pl.ds((wid // (L // ROWS_PER_W)) * L, L)]
    sel0 = (wid % (L // ROWS_PER_W)) * ROWS_PER_W

    def bcast_max(sel):
        return _butterfly(
            jnp.where(lane == sel, mx_chunk, -jnp.inf), jnp.maximum
        )

    rm0 = bcast_max(sel0)
    pltpu.make_async_copy(
        x_hbm.at[base_row, pl.ds(0, H)], buf.at[pl.ds(0, H)], sems.at[2]
    ).wait()
    accs = _sumexp_accs(buf, 0, H, rm0, _zero_accs())
    pltpu.make_async_copy(
        x_hbm.at[base_row, pl.ds(H, H)], buf.at[pl.ds(H, H)], sems.at[3]
    ).wait()
    accs = _sumexp_accs(buf, H, H, rm0, accs)
    acc = jnp.where(lane == 0, 1.0 / _accs_total(accs), jnp.zeros((L,), jnp.float32))

    def row_body(r, acc):
        cur = lax.rem(r, 2)
        nxt = lax.rem(r + 1, 2)

        @pl.when(r + 1 < ROWS_PER_W)
        def _():
            pltpu.make_async_copy(
                x_hbm.at[base_row + r + 1],
                buf.at[pl.ds(nxt * V, V)],
                sems.at[nxt],
            ).start()

        pltpu.make_async_copy(
            x_hbm.at[base_row + r], buf.at[pl.ds(cur * V, V)], sems.at[cur]
        ).wait()

        row_max = bcast_max(sel0 + r)
        sum_exp = _accs_total(_sumexp_accs(buf, cur * V, V, row_max, _zero_accs()))
        return jnp.where(lane == r, 1.0 / sum_exp, acc)

    acc = lax.fori_loop(1, ROWS_PER_W, row_body, acc)

    out_buf[...] = acc
    pltpu.sync_copy(out_buf, out_hbm.at[wid])


def kernel(class_t, dom_res):
    maxes = _tc_row_max(class_t)
    padded = _sc_top1(class_t, maxes)
    return padded[:, :ROWS_PER_W].reshape(-1)


# speedup vs baseline: 1.0385x; 1.0385x over previous
import functools

import jax
import jax.numpy as jnp
from jax import lax
from jax.experimental import pallas as pl
from jax.experimental.pallas import tpu as pltpu
from jax.experimental.pallas import tpu_sc as plsc

B = 128
V = 32768
L = 16
NC = 2
NS = 16
NW = NC * NS
ROWS_PER_W = B // NW
U = 8
CHUNK = U * L
VBLK = 4096


def _tc_max_body(x_ref, o_ref, acc_ref):
    i = pl.program_id(0)
    m = x_ref[:, 0:128]
    for j in range(1, VBLK // 128):
        m = jnp.maximum(m, x_ref[:, j * 128:(j + 1) * 128])

    @pl.when(i == 0)
    def _():
        acc_ref[...] = m

    @pl.when(i > 0)
    def _():
        acc_ref[...] = jnp.maximum(acc_ref[...], m)

    @pl.when(i == V // VBLK - 1)
    def _():
        o_ref[...] = jnp.max(acc_ref[...], axis=1)[None, :]


_tc_row_max = pl.pallas_call(
    _tc_max_body,
    grid=(V // VBLK,),
    in_specs=[pl.BlockSpec((B, VBLK), lambda i: (0, i))],
    out_specs=pl.BlockSpec((1, B), lambda i: (0, 0)),
    out_shape=jax.ShapeDtypeStruct((1, B), jnp.float32),
    scratch_shapes=[pltpu.VMEM((B, 128), jnp.float32)],
)


def _butterfly(v, op):
    lane = lax.iota(jnp.int32, L)
    for k in (8, 4, 2, 1):
        v = op(v, v.at[lane ^ k].get(mode="promise_in_bounds"))
    return v


def _sumexp_accs(buf, base, n, row_max, init):
    @plsc.parallel_loop(0, n, CHUNK, unroll=4, carry=init)
    def ss(off, ss):
        return tuple(
            ss[u] + jnp.exp(buf[pl.ds(base + off + u * L, L)] - row_max)
            for u in range(U)
        )

    return ss


def _accs_total(ss):
    s = ss[0]
    for u in range(1, U):
        s = s + ss[u]
    return _butterfly(s, jnp.add)


def _zero_accs():
    return tuple(jnp.zeros((L,), jnp.float32) for _ in range(U))


@functools.partial(
    pl.kernel,
    mesh=plsc.VectorSubcoreMesh(core_axis_name="c", subcore_axis_name="s"),
    out_type=jax.ShapeDtypeStruct((NW, L), jnp.float32),
    scratch_types=[
        pltpu.VMEM((2 * V,), jnp.float32),
        pltpu.VMEM((B,), jnp.float32),
        pltpu.VMEM((L,), jnp.float32),
        pltpu.SemaphoreType.DMA((4,)),
    ],
)
def _sc_top1(x_hbm, mx_hbm, out_hbm, buf, mx_v, out_buf, sems):
    cid = lax.axis_index("c")
    sid = lax.axis_index("s")
    wid = sid * NC + cid
    base_row = wid * ROWS_PER_W
    H = V // 2

    pltpu.make_async_copy(
        x_hbm.at[base_row, pl.ds(0, H)], buf.at[pl.ds(0, H)], sems.at[2]
    ).start()
    pltpu.make_async_copy(
        x_hbm.at[base_row, pl.ds(H, H)], buf.at[pl.ds(H, H)], sems.at[3]
    ).start()
    pltpu.make_async_copy(
        x_hbm.at[base_row + 1], buf.at[pl.ds(V, V)], sems.at[1]
    ).start()
    pltpu.sync_copy(mx_hbm.at[0], mx_v)

    lane = lax.iota(jnp.int32, L)
    mx_chunk = mx_v[pl.ds((wid // (L // ROWS_PER_W)) * L, L)]
    sel0 = (wid % (L // ROWS_PER_W)) * ROWS_PER_W

    def bcast_max(sel):
        return _butterfly(
            jnp.where(lane == sel, mx_chunk, -jnp.inf), jnp.maximum
        )

    rm0 = bcast_max(sel0)
    pltpu.make_async_copy(
        x_hbm.at[base_row, pl.ds(0, H)], buf.at[pl.ds(0, H)], sems.at[2]
    ).wait()
    accs = _sumexp_accs(buf, 0, H, rm0, _zero_accs())
    pltpu.make_async_copy(
        x_hbm.at[base_row, pl.ds(H, H)], buf.at[pl.ds(H, H)], sems.at[3]
    ).wait()
    accs = _sumexp_accs(buf, H, H, rm0, accs)
    acc = jnp.where(lane == 0, 1.0 / _accs_total(accs), jnp.zeros((L,), jnp.float32))

    def row_body(r, acc):
        cur = lax.rem(r, 2)
        nxt = lax.rem(r + 1, 2)

        @pl.when(r + 1 < ROWS_PER_W)
        def _():
            pltpu.make_async_copy(
                x_hbm.at[base_row + r + 1],
                buf.at[pl.ds(nxt * V, V)],
                sems.at[nxt],
            ).start()

        pltpu.make_async_copy(
            x_hbm.at[base_row + r], buf.at[pl.ds(cur * V, V)], sems.at[cur]
        ).wait()

        row_max = bcast_max(sel0 + r)
        sum_exp = _accs_total(_sumexp_accs(buf, cur * V, V, row_max, _zero_accs()))
        return jnp.where(lane == r, 1.0 / sum_exp, acc)

    acc = lax.fori_loop(1, ROWS_PER_W, row_body, acc)

    out_buf[...] = acc
    pltpu.sync_copy(out_buf, out_hbm.at[wid])


def kernel(class_t, dom_res):
    maxes = _tc_row_max(class_t)
    padded = _sc_top1(class_t, maxes)
    return padded[:, :ROWS_PER_W].reshape(-1)


# speedup vs baseline: 1.0950x; 1.0544x over previous
import functools

import jax
import jax.numpy as jnp
from jax import lax
from jax.experimental import pallas as pl
from jax.experimental.pallas import tpu as pltpu
from jax.experimental.pallas import tpu_sc as plsc

B = 128
V = 32768
L = 16
NC = 2
NS = 16
NW = NC * NS
ROWS_PER_W = B // NW
U = 8
CHUNK = U * L
VBLK = 8192


def _tc_max_body(x_ref, o_ref, acc_ref):
    i = pl.program_id(0)
    m = x_ref[:, 0:128]
    for j in range(1, VBLK // 128):
        m = jnp.maximum(m, x_ref[:, j * 128:(j + 1) * 128])

    @pl.when(i == 0)
    def _():
        acc_ref[...] = m

    @pl.when(i > 0)
    def _():
        acc_ref[...] = jnp.maximum(acc_ref[...], m)

    @pl.when(i == V // VBLK - 1)
    def _():
        o_ref[...] = jnp.max(acc_ref[...], axis=1)[None, :]


_tc_row_max = pl.pallas_call(
    _tc_max_body,
    grid=(V // VBLK,),
    in_specs=[pl.BlockSpec((B, VBLK), lambda i: (0, i))],
    out_specs=pl.BlockSpec((1, B), lambda i: (0, 0)),
    out_shape=jax.ShapeDtypeStruct((1, B), jnp.float32),
    scratch_shapes=[pltpu.VMEM((B, 128), jnp.float32)],
)


def _butterfly(v, op):
    lane = lax.iota(jnp.int32, L)
    for k in (8, 4, 2, 1):
        v = op(v, v.at[lane ^ k].get(mode="promise_in_bounds"))
    return v


def _sumexp_accs(buf, base, n, row_max, init):
    @plsc.parallel_loop(0, n, CHUNK, unroll=4, carry=init)
    def ss(off, ss):
        return tuple(
            ss[u] + jnp.exp(buf[pl.ds(base + off + u * L, L)] - row_max)
            for u in range(U)
        )

    return ss


def _accs_total(ss):
    s = ss[0]
    for u in range(1, U):
        s = s + ss[u]
    return _butterfly(s, jnp.add)


def _zero_accs():
    return tuple(jnp.zeros((L,), jnp.float32) for _ in range(U))


@functools.partial(
    pl.kernel,
    mesh=plsc.VectorSubcoreMesh(core_axis_name="c", subcore_axis_name="s"),
    out_type=jax.ShapeDtypeStruct((NW, L), jnp.float32),
    scratch_types=[
        pltpu.VMEM((2 * V,), jnp.float32),
        pltpu.VMEM((B,), jnp.float32),
        pltpu.VMEM((L,), jnp.float32),
        pltpu.SemaphoreType.DMA((4,)),
    ],
)
def _sc_top1(x_hbm, mx_hbm, out_hbm, buf, mx_v, out_buf, sems):
    cid = lax.axis_index("c")
    sid = lax.axis_index("s")
    wid = sid * NC + cid
    base_row = wid * ROWS_PER_W
    H = V // 2

    pltpu.make_async_copy(
        x_hbm.at[base_row, pl.ds(0, H)], buf.at[pl.ds(0, H)], sems.at[2]
    ).start()
    pltpu.make_async_copy(
        x_hbm.at[base_row, pl.ds(H, H)], buf.at[pl.ds(H, H)], sems.at[3]
    ).start()
    pltpu.make_async_copy(
        x_hbm.at[base_row + 1], buf.at[pl.ds(V, V)], sems.at[1]
    ).start()
    pltpu.sync_copy(mx_hbm.at[0], mx_v)

    lane = lax.iota(jnp.int32, L)
    mx_chunk = mx_v[pl.ds((wid // (L // ROWS_PER_W)) * L, L)]
    sel0 = (wid % (L // ROWS_PER_W)) * ROWS_PER_W

    def bcast_max(sel):
        return _butterfly(
            jnp.where(lane == sel, mx_chunk, -jnp.inf), jnp.maximum
        )

    rm0 = bcast_max(sel0)
    pltpu.make_async_copy(
        x_hbm.at[base_row, pl.ds(0, H)], buf.at[pl.ds(0, H)], sems.at[2]
    ).wait()
    accs = _sumexp_accs(buf, 0, H, rm0, _zero_accs())
    pltpu.make_async_copy(
        x_hbm.at[base_row, pl.ds(H, H)], buf.at[pl.ds(H, H)], sems.at[3]
    ).wait()
    accs = _sumexp_accs(buf, H, H, rm0, accs)
    acc = jnp.where(lane == 0, 1.0 / _accs_total(accs), jnp.zeros((L,), jnp.float32))

    def row_body(r, acc):
        cur = lax.rem(r, 2)
        nxt = lax.rem(r + 1, 2)

        @pl.when(r + 1 < ROWS_PER_W)
        def _():
            pltpu.make_async_copy(
                x_hbm.at[base_row + r + 1],
                buf.at[pl.ds(nxt * V, V)],
                sems.at[nxt],
            ).start()

        pltpu.make_async_copy(
            x_hbm.at[base_row + r], buf.at[pl.ds(cur * V, V)], sems.at[cur]
        ).wait()

        row_max = bcast_max(sel0 + r)
        sum_exp = _accs_total(_sumexp_accs(buf, cur * V, V, row_max, _zero_accs()))
        return jnp.where(lane == r, 1.0 / sum_exp, acc)

    acc = lax.fori_loop(1, ROWS_PER_W, row_body, acc)

    out_buf[...] = acc
    pltpu.sync_copy(out_buf, out_hbm.at[wid])


def kernel(class_t, dom_res):
    maxes = _tc_row_max(class_t)
    padded = _sc_top1(class_t, maxes)
    return padded[:, :ROWS_PER_W].reshape(-1)


# speedup vs baseline: 1.1761x; 1.0740x over previous
import functools

import jax
import jax.numpy as jnp
from jax import lax
from jax.experimental import pallas as pl
from jax.experimental.pallas import tpu as pltpu
from jax.experimental.pallas import tpu_sc as plsc

B = 128
V = 32768
L = 16
NC = 2
NS = 16
NW = NC * NS
ROWS_PER_W = B // NW
U = 8
CHUNK = U * L
VBLK = 8192
HA = 16384
HB = V - HA


def _tc_max_body(x_ref, o_row_ref, o_col_ref, acc_ref):
    i = pl.program_id(0)
    m = x_ref[:, 0:128]
    for j in range(1, VBLK // 128):
        m = jnp.maximum(m, x_ref[:, j * 128:(j + 1) * 128])

    @pl.when(i == 0)
    def _():
        acc_ref[...] = m

    @pl.when(i > 0)
    def _():
        acc_ref[...] = jnp.maximum(acc_ref[...], m)

    @pl.when(i == V // VBLK - 1)
    def _():
        o_col_ref[...] = jnp.max(acc_ref[...], axis=1, keepdims=True)
        o_row_ref[...] = jnp.max(acc_ref[...], axis=1)[None, :]


_tc_row_max = pl.pallas_call(
    _tc_max_body,
    grid=(V // VBLK,),
    in_specs=[pl.BlockSpec((B, VBLK), lambda i: (0, i))],
    out_specs=[
        pl.BlockSpec((1, B), lambda i: (0, 0)),
        pl.BlockSpec((B, 1), lambda i: (0, 0)),
    ],
    out_shape=[
        jax.ShapeDtypeStruct((1, B), jnp.float32),
        jax.ShapeDtypeStruct((B, 1), jnp.float32),
    ],
    scratch_shapes=[pltpu.VMEM((B, 128), jnp.float32)],
)


def _tc_sumexp_body(x_ref, mx_ref, o_ref, acc_ref):
    i = pl.program_id(0)
    mx = mx_ref[...]
    s = jnp.exp(x_ref[:, 0:128] - mx)
    for j in range(1, VBLK // 128):
        s = s + jnp.exp(x_ref[:, j * 128:(j + 1) * 128] - mx)

    @pl.when(i == 0)
    def _():
        acc_ref[...] = s

    @pl.when(i > 0)
    def _():
        acc_ref[...] = acc_ref[...] + s

    @pl.when(i == HA // VBLK - 1)
    def _():
        o_ref[...] = jnp.sum(acc_ref[...], axis=1)[None, :]


_tc_sumexp = pl.pallas_call(
    _tc_sumexp_body,
    grid=(HA // VBLK,),
    in_specs=[
        pl.BlockSpec((B, VBLK), lambda i: (0, i)),
        pl.BlockSpec((B, 1), lambda i: (0, 0)),
    ],
    out_specs=pl.BlockSpec((1, B), lambda i: (0, 0)),
    out_shape=jax.ShapeDtypeStruct((1, B), jnp.float32),
    scratch_shapes=[pltpu.VMEM((B, 128), jnp.float32)],
)


def _butterfly(v, op):
    lane = lax.iota(jnp.int32, L)
    for k in (8, 4, 2, 1):
        v = op(v, v.at[lane ^ k].get(mode="promise_in_bounds"))
    return v


def _sumexp_accs(buf, base, n, row_max, init):
    @plsc.parallel_loop(0, n, CHUNK, unroll=2, carry=init)
    def ss(off, ss):
        return tuple(
            ss[u] + jnp.exp(buf[pl.ds(base + off + u * L, L)] - row_max)
            for u in range(U)
        )

    return ss


def _accs_total(ss):
    s = ss[0]
    for u in range(1, U):
        s = s + ss[u]
    return _butterfly(s, jnp.add)


def _zero_accs():
    return tuple(jnp.zeros((L,), jnp.float32) for _ in range(U))


@functools.partial(
    pl.kernel,
    mesh=plsc.VectorSubcoreMesh(core_axis_name="c", subcore_axis_name="s"),
    out_type=jax.ShapeDtypeStruct((NW, L), jnp.float32),
    scratch_types=[
        pltpu.VMEM((2 * HB,), jnp.float32),
        pltpu.VMEM((B,), jnp.float32),
        pltpu.VMEM((L,), jnp.float32),
        pltpu.SemaphoreType.DMA((4,)),
    ],
)
def _sc_sumexp(x_hbm, mx_hbm, out_hbm, buf, mx_v, out_buf, sems):
    cid = lax.axis_index("c")
    sid = lax.axis_index("s")
    wid = sid * NC + cid
    base_row = wid * ROWS_PER_W
    H = HB // 2

    pltpu.make_async_copy(
        x_hbm.at[base_row, pl.ds(HA, H)], buf.at[pl.ds(0, H)], sems.at[2]
    ).start()
    pltpu.make_async_copy(
        x_hbm.at[base_row, pl.ds(HA + H, H)], buf.at[pl.ds(H, H)], sems.at[3]
    ).start()
    pltpu.make_async_copy(
        x_hbm.at[base_row + 1, pl.ds(HA, HB)], buf.at[pl.ds(HB, HB)], sems.at[1]
    ).start()
    pltpu.sync_copy(mx_hbm.at[0], mx_v)

    lane = lax.iota(jnp.int32, L)
    mx_chunk = mx_v[pl.ds((wid // (L // ROWS_PER_W)) * L, L)]
    sel0 = (wid % (L // ROWS_PER_W)) * ROWS_PER_W

    def bcast_max(sel):
        return _butterfly(
            jnp.where(lane == sel, mx_chunk, -jnp.inf), jnp.maximum
        )

    rm0 = bcast_max(sel0)
    pltpu.make_async_copy(
        x_hbm.at[base_row, pl.ds(HA, H)], buf.at[pl.ds(0, H)], sems.at[2]
    ).wait()
    accs = _sumexp_accs(buf, 0, H, rm0, _zero_accs())
    pltpu.make_async_copy(
        x_hbm.at[base_row, pl.ds(HA + H, H)], buf.at[pl.ds(H, H)], sems.at[3]
    ).wait()
    accs = _sumexp_accs(buf, H, H, rm0, accs)
    acc = jnp.where(
        lane == 0, _accs_total(accs), jnp.zeros((L,), jnp.float32)
    )

    def row_body(r, acc):
        cur = lax.rem(r, 2)
        nxt = lax.rem(r + 1, 2)

        @pl.when(r + 1 < ROWS_PER_W)
        def _():
            pltpu.make_async_copy(
                x_hbm.at[base_row + r + 1, pl.ds(HA, HB)],
                buf.at[pl.ds(nxt * HB, HB)],
                sems.at[nxt],
            ).start()

        pltpu.make_async_copy(
            x_hbm.at[base_row + r, pl.ds(HA, HB)],
            buf.at[pl.ds(cur * HB, HB)],
            sems.at[cur],
        ).wait()

        row_max = bcast_max(sel0 + r)
        sum_exp = _accs_total(
            _sumexp_accs(buf, cur * HB, HB, row_max, _zero_accs())
        )
        return jnp.where(lane == r, sum_exp, acc)

    acc = lax.fori_loop(1, ROWS_PER_W, row_body, acc)

    out_buf[...] = acc
    pltpu.sync_copy(out_buf, out_hbm.at[wid])


def kernel(class_t, dom_res):
    mx_row, mx_col = _tc_row_max(class_t)
    s_tc = _tc_sumexp(class_t, mx_col)
    s_sc_pad = _sc_sumexp(class_t, mx_row)
    s_sc = s_sc_pad[:, :ROWS_PER_W].reshape(B)
    return 1.0 / (s_tc.reshape(B) + s_sc)


# speedup vs baseline: 1.2158x; 1.0338x over previous
import functools

import jax
import jax.numpy as jnp
from jax import lax
from jax.experimental import pallas as pl
from jax.experimental.pallas import tpu as pltpu
from jax.experimental.pallas import tpu_sc as plsc

B = 128
V = 32768
L = 16
NC = 2
NS = 16
NW = NC * NS
ROWS_PER_W = B // NW
U = 8
CHUNK = U * L
VBLK = 8192
VBLK2 = 4096
HA = 20480
HB = V - HA


def _tc_max_body(x_ref, o_row_ref, o_col_ref, acc_ref):
    i = pl.program_id(0)
    m = x_ref[:, 0:128]
    for j in range(1, VBLK // 128):
        m = jnp.maximum(m, x_ref[:, j * 128:(j + 1) * 128])

    @pl.when(i == 0)
    def _():
        acc_ref[...] = m

    @pl.when(i > 0)
    def _():
        acc_ref[...] = jnp.maximum(acc_ref[...], m)

    @pl.when(i == V // VBLK - 1)
    def _():
        o_col_ref[...] = jnp.max(acc_ref[...], axis=1, keepdims=True)
        o_row_ref[...] = jnp.max(acc_ref[...], axis=1)[None, :]


_tc_row_max = pl.pallas_call(
    _tc_max_body,
    grid=(V // VBLK,),
    in_specs=[pl.BlockSpec((B, VBLK), lambda i: (0, i))],
    out_specs=[
        pl.BlockSpec((1, B), lambda i: (0, 0)),
        pl.BlockSpec((B, 1), lambda i: (0, 0)),
    ],
    out_shape=[
        jax.ShapeDtypeStruct((1, B), jnp.float32),
        jax.ShapeDtypeStruct((B, 1), jnp.float32),
    ],
    scratch_shapes=[pltpu.VMEM((B, 128), jnp.float32)],
)


def _tc_sumexp_body(x_ref, mx_ref, o_ref, acc_ref):
    i = pl.program_id(0)
    mx = mx_ref[...]
    s = jnp.exp(x_ref[:, 0:128] - mx)
    for j in range(1, VBLK2 // 128):
        s = s + jnp.exp(x_ref[:, j * 128:(j + 1) * 128] - mx)

    @pl.when(i == 0)
    def _():
        acc_ref[...] = s

    @pl.when(i > 0)
    def _():
        acc_ref[...] = acc_ref[...] + s

    @pl.when(i == HA // VBLK2 - 1)
    def _():
        o_ref[...] = jnp.sum(acc_ref[...], axis=1)[None, :]


_tc_sumexp = pl.pallas_call(
    _tc_sumexp_body,
    grid=(HA // VBLK2,),
    in_specs=[
        pl.BlockSpec((B, VBLK2), lambda i: (0, i)),
        pl.BlockSpec((B, 1), lambda i: (0, 0)),
    ],
    out_specs=pl.BlockSpec((1, B), lambda i: (0, 0)),
    out_shape=jax.ShapeDtypeStruct((1, B), jnp.float32),
    scratch_shapes=[pltpu.VMEM((B, 128), jnp.float32)],
)


def _butterfly(v, op):
    lane = lax.iota(jnp.int32, L)
    for k in (8, 4, 2, 1):
        v = op(v, v.at[lane ^ k].get(mode="promise_in_bounds"))
    return v


def _sumexp_accs(buf, base, n, row_max, init):
    @plsc.parallel_loop(0, n, CHUNK, unroll=2, carry=init)
    def ss(off, ss):
        return tuple(
            ss[u] + jnp.exp(buf[pl.ds(base + off + u * L, L)] - row_max)
            for u in range(U)
        )

    return ss


def _accs_total(ss):
    s = ss[0]
    for u in range(1, U):
        s = s + ss[u]
    return _butterfly(s, jnp.add)


def _zero_accs():
    return tuple(jnp.zeros((L,), jnp.float32) for _ in range(U))


@functools.partial(
    pl.kernel,
    mesh=plsc.VectorSubcoreMesh(core_axis_name="c", subcore_axis_name="s"),
    out_type=jax.ShapeDtypeStruct((NW, L), jnp.float32),
    scratch_types=[
        pltpu.VMEM((2 * HB,), jnp.float32),
        pltpu.VMEM((B,), jnp.float32),
        pltpu.VMEM((L,), jnp.float32),
        pltpu.SemaphoreType.DMA((4,)),
    ],
)
def _sc_sumexp(x_hbm, mx_hbm, out_hbm, buf, mx_v, out_buf, sems):
    cid = lax.axis_index("c")
    sid = lax.axis_index("s")
    wid = sid * NC + cid
    base_row = wid * ROWS_PER_W
    H = HB // 2

    pltpu.make_async_copy(
        x_hbm.at[base_row, pl.ds(HA, H)], buf.at[pl.ds(0, H)], sems.at[2]
    ).start()
    pltpu.make_async_copy(
        x_hbm.at[base_row, pl.ds(HA + H, H)], buf.at[pl.ds(H, H)], sems.at[3]
    ).start()
    pltpu.make_async_copy(
        x_hbm.at[base_row + 1, pl.ds(HA, HB)], buf.at[pl.ds(HB, HB)], sems.at[1]
    ).start()
    pltpu.sync_copy(mx_hbm.at[0], mx_v)

    lane = lax.iota(jnp.int32, L)
    mx_chunk = mx_v[pl.ds((wid // (L // ROWS_PER_W)) * L, L)]
    sel0 = (wid % (L // ROWS_PER_W)) * ROWS_PER_W

    def bcast_max(sel):
        return _butterfly(
            jnp.where(lane == sel, mx_chunk, -jnp.inf), jnp.maximum
        )

    rm0 = bcast_max(sel0)
    pltpu.make_async_copy(
        x_hbm.at[base_row, pl.ds(HA, H)], buf.at[pl.ds(0, H)], sems.at[2]
    ).wait()
    accs = _sumexp_accs(buf, 0, H, rm0, _zero_accs())
    pltpu.make_async_copy(
        x_hbm.at[base_row, pl.ds(HA + H, H)], buf.at[pl.ds(H, H)], sems.at[3]
    ).wait()
    accs = _sumexp_accs(buf, H, H, rm0, accs)
    acc = jnp.where(
        lane == 0, _accs_total(accs), jnp.zeros((L,), jnp.float32)
    )

    def row_body(r, acc):
        cur = lax.rem(r, 2)
        nxt = lax.rem(r + 1, 2)

        @pl.when(r + 1 < ROWS_PER_W)
        def _():
            pltpu.make_async_copy(
                x_hbm.at[base_row + r + 1, pl.ds(HA, HB)],
                buf.at[pl.ds(nxt * HB, HB)],
                sems.at[nxt],
            ).start()

        pltpu.make_async_copy(
            x_hbm.at[base_row + r, pl.ds(HA, HB)],
            buf.at[pl.ds(cur * HB, HB)],
            sems.at[cur],
        ).wait()

        row_max = bcast_max(sel0 + r)
        sum_exp = _accs_total(
            _sumexp_accs(buf, cur * HB, HB, row_max, _zero_accs())
        )
        return jnp.where(lane == r, sum_exp, acc)

    acc = lax.fori_loop(1, ROWS_PER_W, row_body, acc)

    out_buf[...] = acc
    pltpu.sync_copy(out_buf, out_hbm.at[wid])


def kernel(class_t, dom_res):
    mx_row, mx_col = _tc_row_max(class_t)
    s_tc = _tc_sumexp(class_t, mx_col)
    s_sc_pad = _sc_sumexp(class_t, mx_row)
    s_sc = s_sc_pad[:, :ROWS_PER_W].reshape(B)
    return 1.0 / (s_tc.reshape(B) + s_sc)


# speedup vs baseline: 1.2678x; 1.0427x over previous
import functools

import jax
import jax.numpy as jnp
from jax import lax
from jax.experimental import pallas as pl
from jax.experimental.pallas import tpu as pltpu
from jax.experimental.pallas import tpu_sc as plsc

B = 128
V = 32768
L = 16
NC = 2
NS = 16
NW = NC * NS
ROWS_PER_W = B // NW
U = 8
CHUNK = U * L
HA = 20480
HB = V - HA
VBLK = 5120
VBLK2 = 4096


def _tc_max_body(x_ref, o_col_ref, acc_ref):
    i = pl.program_id(0)
    m = x_ref[:, 0:128]
    for j in range(1, VBLK // 128):
        m = jnp.maximum(m, x_ref[:, j * 128:(j + 1) * 128])

    @pl.when(i == 0)
    def _():
        acc_ref[...] = m

    @pl.when(i > 0)
    def _():
        acc_ref[...] = jnp.maximum(acc_ref[...], m)

    @pl.when(i == HA // VBLK - 1)
    def _():
        o_col_ref[...] = jnp.max(acc_ref[...], axis=1, keepdims=True)


_tc_row_max = pl.pallas_call(
    _tc_max_body,
    grid=(HA // VBLK,),
    in_specs=[pl.BlockSpec((B, VBLK), lambda i: (0, i))],
    out_specs=pl.BlockSpec((B, 1), lambda i: (0, 0)),
    out_shape=jax.ShapeDtypeStruct((B, 1), jnp.float32),
    scratch_shapes=[pltpu.VMEM((B, 128), jnp.float32)],
)


def _tc_sumexp_body(x_ref, mx_ref, o_ref, acc_ref):
    i = pl.program_id(0)
    mx = mx_ref[...]
    s = jnp.exp(x_ref[:, 0:128] - mx)
    for j in range(1, VBLK2 // 128):
        s = s + jnp.exp(x_ref[:, j * 128:(j + 1) * 128] - mx)

    @pl.when(i == 0)
    def _():
        acc_ref[...] = s

    @pl.when(i > 0)
    def _():
        acc_ref[...] = acc_ref[...] + s

    @pl.when(i == HA // VBLK2 - 1)
    def _():
        o_ref[...] = jnp.sum(acc_ref[...], axis=1, keepdims=True)


_tc_sumexp = pl.pallas_call(
    _tc_sumexp_body,
    grid=(HA // VBLK2,),
    in_specs=[
        pl.BlockSpec((B, VBLK2), lambda i: (0, i)),
        pl.BlockSpec((B, 1), lambda i: (0, 0)),
    ],
    out_specs=pl.BlockSpec((B, 1), lambda i: (0, 0)),
    out_shape=jax.ShapeDtypeStruct((B, 1), jnp.float32),
    scratch_shapes=[pltpu.VMEM((B, 128), jnp.float32)],
)


def _butterfly(v, op):
    lane = lax.iota(jnp.int32, L)
    for k in (8, 4, 2, 1):
        v = op(v, v.at[lane ^ k].get(mode="promise_in_bounds"))
    return v


def _max_accs(buf, base, n, init):
    @plsc.parallel_loop(0, n, CHUNK, unroll=2, carry=init)
    def ms(off, ms):
        return tuple(
            jnp.maximum(ms[u], buf[pl.ds(base + off + u * L, L)])
            for u in range(U)
        )

    return ms


def _sumexp_accs(buf, base, n, row_max, init):
    @plsc.parallel_loop(0, n, CHUNK, unroll=2, carry=init)
    def ss(off, ss):
        return tuple(
            ss[u] + jnp.exp(buf[pl.ds(base + off + u * L, L)] - row_max)
            for u in range(U)
        )

    return ss


def _tree_total(ss, op):
    s = ss[0]
    for u in range(1, U):
        s = op(s, ss[u])
    return _butterfly(s, op)


def _minf_accs():
    return tuple(jnp.full((L,), -jnp.inf, jnp.float32) for _ in range(U))


def _zero_accs():
    return tuple(jnp.zeros((L,), jnp.float32) for _ in range(U))


@functools.partial(
    pl.kernel,
    mesh=plsc.VectorSubcoreMesh(core_axis_name="c", subcore_axis_name="s"),
    out_type=[
        jax.ShapeDtypeStruct((NW, L), jnp.float32),
        jax.ShapeDtypeStruct((NW, L), jnp.float32),
    ],
    scratch_types=[
        pltpu.VMEM((2 * HB,), jnp.float32),
        pltpu.VMEM((L,), jnp.float32),
        pltpu.VMEM((L,), jnp.float32),
        pltpu.SemaphoreType.DMA((4,)),
    ],
)
def _sc_local(x_hbm, s_hbm, m_hbm, buf, s_buf, m_buf, sems):
    cid = lax.axis_index("c")
    sid = lax.axis_index("s")
    wid = sid * NC + cid
    base_row = wid * ROWS_PER_W
    H = HB // 2

    pltpu.make_async_copy(
        x_hbm.at[base_row, pl.ds(HA, H)], buf.at[pl.ds(0, H)], sems.at[2]
    ).start()
    pltpu.make_async_copy(
        x_hbm.at[base_row, pl.ds(HA + H, H)], buf.at[pl.ds(H, H)], sems.at[3]
    ).start()
    pltpu.make_async_copy(
        x_hbm.at[base_row + 1, pl.ds(HA, HB)], buf.at[pl.ds(HB, HB)], sems.at[1]
    ).start()

    lane = lax.iota(jnp.int32, L)

    pltpu.make_async_copy(
        x_hbm.at[base_row, pl.ds(HA, H)], buf.at[pl.ds(0, H)], sems.at[2]
    ).wait()
    mac = _max_accs(buf, 0, H, _minf_accs())
    pltpu.make_async_copy(
        x_hbm.at[base_row, pl.ds(HA + H, H)], buf.at[pl.ds(H, H)], sems.at[3]
    ).wait()
    rm0 = _tree_total(_max_accs(buf, H, H, mac), jnp.maximum)
    s0 = _tree_total(_sumexp_accs(buf, 0, HB, rm0, _zero_accs()), jnp.add)
    zero = jnp.zeros((L,), jnp.float32)
    s_acc = jnp.where(lane == 0, s0, zero)
    m_acc = jnp.where(lane == 0, rm0, zero)

    def row_body(r, carry):
        s_acc, m_acc = carry
        cur = lax.rem(r, 2)
        nxt = lax.rem(r + 1, 2)

        @pl.when(r + 1 < ROWS_PER_W)
        def _():
            pltpu.make_async_copy(
                x_hbm.at[base_row + r + 1, pl.ds(HA, HB)],
                buf.at[pl.ds(nxt * HB, HB)],
                sems.at[nxt],
            ).start()

        pltpu.make_async_copy(
            x_hbm.at[base_row + r, pl.ds(HA, HB)],
            buf.at[pl.ds(cur * HB, HB)],
            sems.at[cur],
        ).wait()

        rm = _tree_total(
            _max_accs(buf, cur * HB, HB, _minf_accs()), jnp.maximum
        )
        s = _tree_total(
            _sumexp_accs(buf, cur * HB, HB, rm, _zero_accs()), jnp.add
        )
        return (
            jnp.where(lane == r, s, s_acc),
            jnp.where(lane == r, rm, m_acc),
        )

    s_acc, m_acc = lax.fori_loop(1, ROWS_PER_W, row_body, (s_acc, m_acc))

    s_buf[...] = s_acc
    m_buf[...] = m_acc
    pltpu.sync_copy(s_buf, s_hbm.at[wid])
    pltpu.sync_copy(m_buf, m_hbm.at[wid])


def kernel(class_t, dom_res):
    s_sc_pad, m_sc_pad = _sc_local(class_t)
    mA = _tc_row_max(class_t)
    sA = _tc_sumexp(class_t, mA)
    sB = s_sc_pad[:, :ROWS_PER_W].reshape(B)
    mB = m_sc_pad[:, :ROWS_PER_W].reshape(B)
    mA = mA.reshape(B)
    sA = sA.reshape(B)
    mm = jnp.maximum(mA, mB)
    return 1.0 / (sA * jnp.exp(mA - mm) + sB * jnp.exp(mB - mm))
